# Initial kernel scaffold; baseline (speedup 1.0000x reference)
import functools

import jax
import jax.numpy as jnp
from jax import lax
from jax.experimental import pallas as pl
from jax.experimental.pallas import tpu as pltpu
from jax.experimental.pallas import tpu_sc as plsc

N = 10000
D = 128
NC = 2
NS = 16
NW = NC * NS
BLK = 128
N_PAD = 10240
ZR = N_PAD // NS
ZCH = 128
EPS = 1e-5
INV_SQRT1P = 1.0 / (1.0 + EPS) ** 0.5

_mesh = plsc.VectorSubcoreMesh(core_axis_name="c", subcore_axis_name="s")


def _nb(e_tot):
    return (e_tot + NW * BLK - 1) // (NW * BLK)


def _sc_degree(nb):
    def body(dst_hbm, zeros_hbm, ones_hbm, out_hbm, dst_v, zb_v, ones_v, wb_v, acc_sh):
        c = lax.axis_index("c")
        s = lax.axis_index("s")
        w = s * NC + c
        pltpu.sync_copy(zeros_hbm, zb_v)
        pltpu.sync_copy(ones_hbm, ones_v)
        pltpu.sync_copy(dst_hbm.at[w], dst_v)

        def zloop(i, _):
            pltpu.sync_copy(zb_v, acc_sh.at[pl.ds(s * ZR + i * ZCH, ZCH)])
            return 0
        lax.fori_loop(0, ZR // ZCH, zloop, 0)
        plsc.subcore_barrier()

        def eloop(j, _):
            pltpu.sync_copy(ones_v, acc_sh.at[dst_v.at[j]], add=True)
            return 0
        lax.fori_loop(0, nb, eloop, 0)
        plsc.subcore_barrier()

        def wloop(i, _):
            r = s * ZR + i * ZCH
            pltpu.sync_copy(acc_sh.at[pl.ds(r, ZCH)], wb_v)
            pltpu.sync_copy(wb_v, out_hbm.at[c].at[pl.ds(r, ZCH)])
            return 0
        lax.fori_loop(0, ZR // ZCH, wloop, 0)

    return pl.kernel(
        body,
        mesh=_mesh,
        out_type=jax.ShapeDtypeStruct((NC, N_PAD, 16), jnp.float32),
        scratch_types=[
            pltpu.VMEM((nb, BLK), jnp.int32),
            pltpu.VMEM((ZCH, 16), jnp.float32),
            pltpu.VMEM((BLK, 16), jnp.float32),
            pltpu.VMEM((ZCH, 16), jnp.float32),
            pltpu.VMEM_SHARED((N_PAD, 16), jnp.float32),
        ],
    )


def _sc_propagate(nb):
    def body(y_hbm, src_hbm, dst_hbm, zeros_hbm, out_hbm,
             src_v, dst_v, rows_v, zb_v, sem, acc_sh):
        c = lax.axis_index("c")
        s = lax.axis_index("s")
        w = s * NC + c
        pltpu.sync_copy(zeros_hbm, zb_v)
        pltpu.sync_copy(src_hbm.at[w], src_v)
        pltpu.sync_copy(dst_hbm.at[w], dst_v)

        def zloop(i, _):
            pltpu.sync_copy(zb_v, acc_sh.at[pl.ds(s * ZR + i * ZCH, ZCH)])
            return 0
        lax.fori_loop(0, ZR // ZCH, zloop, 0)
        plsc.subcore_barrier()

        def eloop(j, _):
            pltpu.async_copy(y_hbm.at[src_v.at[j]], rows_v, sem).wait()
            pltpu.sync_copy(rows_v, acc_sh.at[dst_v.at[j]], add=True)
            return 0
        lax.fori_loop(0, nb, eloop, 0)
        plsc.subcore_barrier()

        def wloop(i, _):
            r = s * ZR + i * ZCH
            pltpu.sync_copy(acc_sh.at[pl.ds(r, ZCH)], zb_v)
            pltpu.sync_copy(zb_v, out_hbm.at[c].at[pl.ds(r, ZCH)])
            return 0
        lax.fori_loop(0, ZR // ZCH, wloop, 0)

    return pl.kernel(
        body,
        mesh=_mesh,
        out_type=jax.ShapeDtypeStruct((NC, N_PAD, D), jnp.float32),
        scratch_types=[
            pltpu.VMEM((nb, BLK), jnp.int32),
            pltpu.VMEM((nb, BLK), jnp.int32),
            pltpu.VMEM((BLK, D), jnp.float32),
            pltpu.VMEM((ZCH, D), jnp.float32),
            pltpu.SemaphoreType.DMA,
            pltpu.VMEM_SHARED((N_PAD, D), jnp.float32),
        ],
    )


RB = 2048


def _dinv_of(d0, d1):
    deg = d0[:, 0:1] + d1[:, 0:1]
    return jnp.where(deg > 0.0, lax.rsqrt(deg), 0.0)


def _tc_y1(x_ref, w_ref, d0_ref, d1_ref, y_ref):
    dinv = _dinv_of(d0_ref[...], d1_ref[...])
    xw = jnp.dot(x_ref[...], w_ref[...], preferred_element_type=jnp.float32)
    y_ref[...] = xw * dinv


def _tc_y2(a0_ref, a1_ref, d0_ref, d1_ref, b1_ref, g_ref, be_ref, w2_ref, y_ref):
    dinv = _dinv_of(d0_ref[...], d1_ref[...])
    h = (a0_ref[...] + a1_ref[...]) * dinv + b1_ref[...]
    h = h * (g_ref[...] * INV_SQRT1P) + be_ref[...]
    h = jnp.maximum(h, 0.0)
    y_ref[...] = jnp.dot(h, w2_ref[...], preferred_element_type=jnp.float32) * dinv


def _tc_out(a0_ref, a1_ref, d0_ref, d1_ref, b2_ref, o_ref):
    dinv = _dinv_of(d0_ref[...], d1_ref[...])
    o_ref[...] = (a0_ref[...] + a1_ref[...]) * dinv + b2_ref[...]


def _row_spec(i_dim):
    return pl.BlockSpec((RB, i_dim), lambda i: (i, 0))


def _bcast_spec(shape):
    return pl.BlockSpec(shape, lambda i: (0, 0))


def kernel(x, edge_index, W1, b1, gamma1, beta1, W2, b2):
    e = edge_index.shape[1]
    e_tot = e + N
    nb = _nb(e_tot)
    e_pad = NW * nb * BLK - e_tot

    loop = jnp.arange(N, dtype=jnp.int32)
    padi = jnp.arange(e_pad, dtype=jnp.int32)
    src3 = jnp.concatenate(
        [edge_index[0].astype(jnp.int32), loop, N + (padi % 64)]).reshape(NW, nb, BLK)
    dst3 = jnp.concatenate(
        [edge_index[1].astype(jnp.int32), loop, N + 64 + (padi % 64)]).reshape(NW, nb, BLK)
    x_pad = jnp.pad(x, ((0, N_PAD - N), (0, 0)))

    zeros16 = jnp.zeros((ZCH, 16), jnp.float32)
    ones16 = jnp.ones((BLK, 16), jnp.float32)
    zerosD = jnp.zeros((ZCH, D), jnp.float32)

    deg = _sc_degree(nb)(dst3, zeros16, ones16)
    d0, d1 = deg[0], deg[1]

    grid = (N_PAD // RB,)
    y1 = pl.pallas_call(
        _tc_y1,
        grid=grid,
        in_specs=[_row_spec(D), _bcast_spec((D, D)), _row_spec(16), _row_spec(16)],
        out_specs=_row_spec(D),
        out_shape=jax.ShapeDtypeStruct((N_PAD, D), jnp.float32),
    )(x_pad, W1, d0, d1)

    prop = _sc_propagate(nb)
    acc1 = prop(y1, src3, dst3, zerosD)

    y2 = pl.pallas_call(
        _tc_y2,
        grid=grid,
        in_specs=[_row_spec(D), _row_spec(D), _row_spec(16), _row_spec(16),
                  _bcast_spec((1, D)), _bcast_spec((1, D)), _bcast_spec((1, D)),
                  _bcast_spec((D, D))],
        out_specs=_row_spec(D),
        out_shape=jax.ShapeDtypeStruct((N_PAD, D), jnp.float32),
    )(acc1[0], acc1[1], d0, d1, b1.reshape(1, D), gamma1.reshape(1, D),
      beta1.reshape(1, D), W2)

    acc2 = prop(y2, src3, dst3, zerosD)

    out = pl.pallas_call(
        _tc_out,
        grid=grid,
        in_specs=[_row_spec(D), _row_spec(D), _row_spec(16), _row_spec(16),
                  _bcast_spec((1, D))],
        out_specs=_row_spec(D),
        out_shape=jax.ShapeDtypeStruct((N_PAD, D), jnp.float32),
    )(acc2[0], acc2[1], d0, d1, b2.reshape(1, D))

    return out[:N]


# speedup vs baseline: 14.3904x; 14.3904x over previous
import functools

import jax
import jax.numpy as jnp
from jax import lax
from jax.experimental import pallas as pl
from jax.experimental.pallas import tpu as pltpu
from jax.experimental.pallas import tpu_sc as plsc

N = 10000
D = 128
NC = 2
NS = 16
NW = NC * NS
BLK = 128
N_PAD = 10240
ZR = N_PAD // NS
ZCH = 128
EPS = 1e-5
INV_SQRT1P = 1.0 / (1.0 + EPS) ** 0.5

_f32 = jnp.float32


@functools.cache
def _mesh():
    return plsc.VectorSubcoreMesh(
        core_axis_name="c", subcore_axis_name="s", num_cores=NC, num_subcores=NS)


def _nb(e_tot):
    return (e_tot + NW * BLK - 1) // (NW * BLK)


def _sc_degree(nb):
    def body(dst_hbm, zeros_hbm, ones_hbm, out_hbm, dst_v, buf_v, acc_sh):
        c = lax.axis_index("c")
        s = lax.axis_index("s")
        w = s * NC + c
        pltpu.sync_copy(zeros_hbm, buf_v)

        def zloop(i, _):
            pltpu.sync_copy(buf_v, acc_sh.at[pl.ds(s * ZR + i * ZCH, ZCH)])
            return 0
        lax.fori_loop(0, ZR // ZCH, zloop, 0)
        pltpu.sync_copy(ones_hbm, buf_v)
        plsc.subcore_barrier()

        def eloop(j, _):
            pltpu.sync_copy(dst_hbm.at[pl.ds((w * nb + j) * BLK, BLK)], dst_v)
            pltpu.sync_copy(buf_v, acc_sh.at[dst_v], add=True)
            return 0
        lax.fori_loop(0, nb, eloop, 0)
        plsc.subcore_barrier()

        def wloop(i, _):
            r = s * ZR + i * ZCH
            pltpu.sync_copy(acc_sh.at[pl.ds(r, ZCH)], buf_v)
            pltpu.sync_copy(buf_v, out_hbm.at[c, pl.ds(r, ZCH)])
            return 0
        lax.fori_loop(0, ZR // ZCH, wloop, 0)

    return pl.kernel(
        body,
        mesh=_mesh(),
        out_type=jax.ShapeDtypeStruct((NC, N_PAD, D), _f32),
        scratch_types=[
            pltpu.VMEM((BLK,), jnp.int32),
            pltpu.VMEM((BLK, D), _f32),
            pltpu.VMEM_SHARED((N_PAD, D), _f32),
        ],
    )


def _sc_propagate(nb):
    def body(y_hbm, src_hbm, dst_hbm, zeros_hbm, out_hbm,
             src_v, dst_v, rows_v, sem, acc_sh):
        c = lax.axis_index("c")
        s = lax.axis_index("s")
        w = s * NC + c
        pltpu.sync_copy(zeros_hbm, rows_v)

        def zloop(i, _):
            pltpu.sync_copy(rows_v, acc_sh.at[pl.ds(s * ZR + i * ZCH, ZCH)])
            return 0
        lax.fori_loop(0, ZR // ZCH, zloop, 0)
        plsc.subcore_barrier()

        def eloop(j, _):
            pltpu.sync_copy(src_hbm.at[pl.ds((w * nb + j) * BLK, BLK)], src_v)
            pltpu.sync_copy(dst_hbm.at[pl.ds((w * nb + j) * BLK, BLK)], dst_v)
            pltpu.async_copy(y_hbm.at[src_v], rows_v, sem).wait()
            pltpu.sync_copy(rows_v, acc_sh.at[dst_v], add=True)
            return 0
        lax.fori_loop(0, nb, eloop, 0)
        plsc.subcore_barrier()

        def wloop(i, _):
            r = s * ZR + i * ZCH
            pltpu.sync_copy(acc_sh.at[pl.ds(r, ZCH)], rows_v)
            pltpu.sync_copy(rows_v, out_hbm.at[c, pl.ds(r, ZCH)])
            return 0
        lax.fori_loop(0, ZR // ZCH, wloop, 0)

    return pl.kernel(
        body,
        mesh=_mesh(),
        out_type=jax.ShapeDtypeStruct((NC, N_PAD, D), _f32),
        scratch_types=[
            pltpu.VMEM((BLK,), jnp.int32),
            pltpu.VMEM((BLK,), jnp.int32),
            pltpu.VMEM((BLK, D), _f32),
            pltpu.SemaphoreType.DMA,
            pltpu.VMEM_SHARED((N_PAD, D), _f32),
        ],
    )


RB = 2048


def _dinv_of(d0, d1):
    deg = d0[:, 0:1] + d1[:, 0:1]
    return jnp.where(deg > 0.0, lax.rsqrt(deg), 0.0)


def _tc_y1(x_ref, w_ref, d0_ref, d1_ref, y_ref):
    dinv = _dinv_of(d0_ref[...], d1_ref[...])
    xw = jnp.dot(x_ref[...], w_ref[...], preferred_element_type=_f32)
    y_ref[...] = xw * dinv


def _tc_y2(a0_ref, a1_ref, d0_ref, d1_ref, b1_ref, g_ref, be_ref, w2_ref, y_ref):
    dinv = _dinv_of(d0_ref[...], d1_ref[...])
    h = (a0_ref[...] + a1_ref[...]) * dinv + b1_ref[...]
    h = h * (g_ref[...] * INV_SQRT1P) + be_ref[...]
    h = jnp.maximum(h, 0.0)
    y_ref[...] = jnp.dot(h, w2_ref[...], preferred_element_type=_f32) * dinv


def _tc_out(a0_ref, a1_ref, d0_ref, d1_ref, b2_ref, o_ref):
    dinv = _dinv_of(d0_ref[...], d1_ref[...])
    o_ref[...] = (a0_ref[...] + a1_ref[...]) * dinv + b2_ref[...]


def _row_spec(i_dim):
    return pl.BlockSpec((RB, i_dim), lambda i: (i, 0))


def _bcast_spec(shape):
    return pl.BlockSpec(shape, lambda i: (0, 0))


def kernel(x, edge_index, W1, b1, gamma1, beta1, W2, b2):
    e = edge_index.shape[1]
    e_tot = e + N
    nb = _nb(e_tot)
    e_pad = NW * nb * BLK - e_tot

    loop = jnp.arange(N, dtype=jnp.int32)
    padi = jnp.arange(e_pad, dtype=jnp.int32)
    src_f = jnp.concatenate([edge_index[0].astype(jnp.int32), loop, N + (padi % 64)])
    dst_f = jnp.concatenate([edge_index[1].astype(jnp.int32), loop, N + 64 + (padi % 64)])
    x_pad = jnp.pad(x, ((0, N_PAD - N), (0, 0)))

    zerosD = jnp.zeros((ZCH, D), _f32)
    onesD = jnp.ones((BLK, D), _f32)

    deg = _sc_degree(nb)(dst_f, zerosD, onesD)
    d0, d1 = deg[0], deg[1]

    grid = (N_PAD // RB,)
    y1 = pl.pallas_call(
        _tc_y1,
        grid=grid,
        in_specs=[_row_spec(D), _bcast_spec((D, D)), _row_spec(D), _row_spec(D)],
        out_specs=_row_spec(D),
        out_shape=jax.ShapeDtypeStruct((N_PAD, D), _f32),
    )(x_pad, W1, d0, d1)

    prop = _sc_propagate(nb)
    acc1 = prop(y1, src_f, dst_f, zerosD)

    y2 = pl.pallas_call(
        _tc_y2,
        grid=grid,
        in_specs=[_row_spec(D), _row_spec(D), _row_spec(D), _row_spec(D),
                  _bcast_spec((1, D)), _bcast_spec((1, D)), _bcast_spec((1, D)),
                  _bcast_spec((D, D))],
        out_specs=_row_spec(D),
        out_shape=jax.ShapeDtypeStruct((N_PAD, D), _f32),
    )(acc1[0], acc1[1], d0, d1, b1.reshape(1, D), gamma1.reshape(1, D),
      beta1.reshape(1, D), W2)

    acc2 = prop(y2, src_f, dst_f, zerosD)

    out = pl.pallas_call(
        _tc_out,
        grid=grid,
        in_specs=[_row_spec(D), _row_spec(D), _row_spec(D), _row_spec(D),
                  _bcast_spec((1, D))],
        out_specs=_row_spec(D),
        out_shape=jax.ShapeDtypeStruct((N_PAD, D), _f32),
    )(acc2[0], acc2[1], d0, d1, b2.reshape(1, D))

    return out[:N]


# speedup vs baseline: 23.5018x; 1.6332x over previous
import functools

import jax
import jax.numpy as jnp
from jax import lax
from jax.experimental import pallas as pl
from jax.experimental.pallas import tpu as pltpu
from jax.experimental.pallas import tpu_sc as plsc

N = 10000
D = 128
NC = 2
NS = 16
NW = NC * NS
BLK = 128
N_PAD = 10240
ZR = N_PAD // NS
ZCH = 128
EPS = 1e-5
INV_SQRT1P = 1.0 / (1.0 + EPS) ** 0.5

_f32 = jnp.float32


@functools.cache
def _mesh():
    return plsc.VectorSubcoreMesh(
        core_axis_name="c", subcore_axis_name="s", num_cores=NC, num_subcores=NS)


def _nb(e_tot):
    return (e_tot + NW * BLK - 1) // (NW * BLK)


def _sc_degree(nb):
    def body(dst_hbm, zeros_hbm, ones_hbm, out_hbm, dst_v, buf_v, acc_sh):
        c = lax.axis_index("c")
        s = lax.axis_index("s")
        w = s * NC + c
        pltpu.sync_copy(zeros_hbm, buf_v)

        def zloop(i, _):
            pltpu.sync_copy(buf_v, acc_sh.at[pl.ds(s * ZR + i * ZCH, ZCH)])
            return 0
        lax.fori_loop(0, ZR // ZCH, zloop, 0)
        pltpu.sync_copy(ones_hbm, buf_v)
        plsc.subcore_barrier()

        def eloop(j, _):
            pltpu.sync_copy(dst_hbm.at[pl.ds((w * nb + j) * BLK, BLK)], dst_v)
            pltpu.sync_copy(buf_v, acc_sh.at[dst_v], add=True)
            return 0
        lax.fori_loop(0, nb, eloop, 0)
        plsc.subcore_barrier()

        def wloop(i, _):
            r = s * ZR + i * ZCH
            pltpu.sync_copy(acc_sh.at[pl.ds(r, ZCH)], buf_v)
            pltpu.sync_copy(buf_v, out_hbm.at[c, pl.ds(r, ZCH)])
            return 0
        lax.fori_loop(0, ZR // ZCH, wloop, 0)

    return pl.kernel(
        body,
        mesh=_mesh(),
        out_type=jax.ShapeDtypeStruct((NC, N_PAD, D), _f32),
        scratch_types=[
            pltpu.VMEM((BLK,), jnp.int32),
            pltpu.VMEM((BLK, D), _f32),
            pltpu.VMEM_SHARED((N_PAD, D), _f32),
        ],
    )


def _sc_propagate(nb):
    def body(y_hbm, src_hbm, dst_hbm, zeros_hbm, out_hbm,
             sb0, sb1, sb2, sb3, db0, db1, db2, db3, r0, r1,
             is0, is1, is2, is3, id0, id1, id2, id3, g0, g1, sc, acc_sh):
        c = lax.axis_index("c")
        s = lax.axis_index("s")
        w = s * NC + c
        rows = (r0, r1)
        gsem = (g0, g1)
        srcb = (sb0, sb1, sb2, sb3)
        dstb = (db0, db1, db2, db3)
        isem = (is0, is1, is2, is3)
        idsem = (id0, id1, id2, id3)

        def s_sl(j):
            return src_hbm.at[pl.ds((w * nb + j) * BLK, BLK)]

        def d_sl(j):
            return dst_hbm.at[pl.ds((w * nb + j) * BLK, BLK)]

        def idx_issue(j, b4):
            pltpu.async_copy(s_sl(j), srcb[b4], isem[b4])
            pltpu.async_copy(d_sl(j), dstb[b4], idsem[b4])

        pltpu.sync_copy(zeros_hbm, r0)

        def zloop(i, _):
            pltpu.sync_copy(r0, acc_sh.at[pl.ds(s * ZR + i * ZCH, ZCH)])
            return 0
        lax.fori_loop(0, ZR // ZCH, zloop, 0)
        plsc.subcore_barrier()

        for j in range(4):
            idx_issue(j, j)
        for j in range(2):
            pltpu.make_async_copy(s_sl(j), srcb[j], isem[j]).wait()
            pltpu.async_copy(y_hbm.at[srcb[j]], rows[j], gsem[j])

        def eloop(g, _):
            for b4 in range(4):
                j = g * 4 + b4
                b2 = b4 % 2
                pltpu.make_async_copy(y_hbm.at[srcb[b4]], rows[b2], gsem[b2]).wait()
                pltpu.make_async_copy(d_sl(j), dstb[b4], idsem[b4]).wait()
                pltpu.async_copy(rows[b2], acc_sh.at[dstb[b4]], sc, add=True)

                @pl.when(j + 2 < nb)
                def _():
                    b4n = (b4 + 2) % 4
                    pltpu.make_async_copy(rows[b2], acc_sh.at[dstb[b4]], sc).wait()
                    pltpu.make_async_copy(s_sl(j + 2), srcb[b4n], isem[b4n]).wait()
                    pltpu.async_copy(y_hbm.at[srcb[b4n]], rows[b2], gsem[b2])

                @pl.when(j + 4 < nb)
                def _():
                    idx_issue(j + 4, b4)
            return 0
        lax.fori_loop(0, nb // 4, eloop, 0)
        for j in range(nb - 2, nb):
            pltpu.make_async_copy(rows[j % 2], acc_sh.at[dstb[j % 4]], sc).wait()
        plsc.subcore_barrier()

        def wloop(i, _):
            r = s * ZR + i * ZCH
            pltpu.sync_copy(acc_sh.at[pl.ds(r, ZCH)], r0)
            pltpu.sync_copy(r0, out_hbm.at[c, pl.ds(r, ZCH)])
            return 0
        lax.fori_loop(0, ZR // ZCH, wloop, 0)

    return pl.kernel(
        body,
        mesh=_mesh(),
        out_type=jax.ShapeDtypeStruct((NC, N_PAD, D), _f32),
        scratch_types=(
            [pltpu.VMEM((BLK,), jnp.int32)] * 8
            + [pltpu.VMEM((BLK, D), _f32)] * 2
            + [pltpu.SemaphoreType.DMA] * 11
            + [pltpu.VMEM_SHARED((N_PAD, D), _f32)]
        ),
    )


RB = 2048


def _dinv_of(d0, d1):
    deg = d0[:, 0:1] + d1[:, 0:1]
    return jnp.where(deg > 0.0, lax.rsqrt(deg), 0.0)


def _tc_y1(x_ref, w_ref, d0_ref, d1_ref, y_ref):
    dinv = _dinv_of(d0_ref[...], d1_ref[...])
    xw = jnp.dot(x_ref[...], w_ref[...], preferred_element_type=_f32)
    y_ref[...] = xw * dinv


def _tc_y2(a0_ref, a1_ref, d0_ref, d1_ref, b1_ref, g_ref, be_ref, w2_ref, y_ref):
    dinv = _dinv_of(d0_ref[...], d1_ref[...])
    h = (a0_ref[...] + a1_ref[...]) * dinv + b1_ref[...]
    h = h * (g_ref[...] * INV_SQRT1P) + be_ref[...]
    h = jnp.maximum(h, 0.0)
    y_ref[...] = jnp.dot(h, w2_ref[...], preferred_element_type=_f32) * dinv


def _tc_out(a0_ref, a1_ref, d0_ref, d1_ref, b2_ref, o_ref):
    dinv = _dinv_of(d0_ref[...], d1_ref[...])
    o_ref[...] = (a0_ref[...] + a1_ref[...]) * dinv + b2_ref[...]


def _row_spec(i_dim):
    return pl.BlockSpec((RB, i_dim), lambda i: (i, 0))


def _bcast_spec(shape):
    return pl.BlockSpec(shape, lambda i: (0, 0))


def kernel(x, edge_index, W1, b1, gamma1, beta1, W2, b2):
    e = edge_index.shape[1]
    e_tot = e + N
    nb = _nb(e_tot)
    nb = ((nb + 3) // 4) * 4
    e_pad = NW * nb * BLK - e_tot

    loop = jnp.arange(N, dtype=jnp.int32)
    padi = jnp.arange(e_pad, dtype=jnp.int32)
    src_f = jnp.concatenate([edge_index[0].astype(jnp.int32), loop, N + (padi % 64)])
    dst_f = jnp.concatenate([edge_index[1].astype(jnp.int32), loop, N + 64 + (padi % 64)])
    x_pad = jnp.pad(x, ((0, N_PAD - N), (0, 0)))

    zerosD = jnp.zeros((ZCH, D), _f32)
    onesD = jnp.ones((BLK, D), _f32)

    deg = _sc_degree(nb)(dst_f, zerosD, onesD)
    d0, d1 = deg[0], deg[1]

    grid = (N_PAD // RB,)
    y1 = pl.pallas_call(
        _tc_y1,
        grid=grid,
        in_specs=[_row_spec(D), _bcast_spec((D, D)), _row_spec(D), _row_spec(D)],
        out_specs=_row_spec(D),
        out_shape=jax.ShapeDtypeStruct((N_PAD, D), _f32),
    )(x_pad, W1, d0, d1)

    prop = _sc_propagate(nb)
    acc1 = prop(y1, src_f, dst_f, zerosD)

    y2 = pl.pallas_call(
        _tc_y2,
        grid=grid,
        in_specs=[_row_spec(D), _row_spec(D), _row_spec(D), _row_spec(D),
                  _bcast_spec((1, D)), _bcast_spec((1, D)), _bcast_spec((1, D)),
                  _bcast_spec((D, D))],
        out_specs=_row_spec(D),
        out_shape=jax.ShapeDtypeStruct((N_PAD, D), _f32),
    )(acc1[0], acc1[1], d0, d1, b1.reshape(1, D), gamma1.reshape(1, D),
      beta1.reshape(1, D), W2)

    acc2 = prop(y2, src_f, dst_f, zerosD)

    out = pl.pallas_call(
        _tc_out,
        grid=grid,
        in_specs=[_row_spec(D), _row_spec(D), _row_spec(D), _row_spec(D),
                  _bcast_spec((1, D))],
        out_specs=_row_spec(D),
        out_shape=jax.ShapeDtypeStruct((N_PAD, D), _f32),
    )(acc2[0], acc2[1], d0, d1, b2.reshape(1, D))

    return out[:N]


# speedup vs baseline: 26.2134x; 1.1154x over previous
import functools

import jax
import jax.numpy as jnp
from jax import lax
from jax.experimental import pallas as pl
from jax.experimental.pallas import tpu as pltpu
from jax.experimental.pallas import tpu_sc as plsc

N = 10000
D = 128
NC = 2
NS = 16
NW = NC * NS
BLK = 112
N_PAD = 10240
ZR = N_PAD // NS
ZCH = 128
EPS = 1e-5
INV_SQRT1P = 1.0 / (1.0 + EPS) ** 0.5

_f32 = jnp.float32


@functools.cache
def _mesh():
    return plsc.VectorSubcoreMesh(
        core_axis_name="c", subcore_axis_name="s", num_cores=NC, num_subcores=NS)


def _nb(e_tot):
    return (e_tot + NW * BLK - 1) // (NW * BLK)


def _sc_degree(nb):
    def body(dst_hbm, zeros_hbm, ones_hbm, out_hbm,
             db0, db1, db2, db3, db4, db5, db6, db7, buf_v, ones_v,
             id0, id1, id2, id3, id4, id5, id6, id7, sa, sb, acc_sh):
        c = lax.axis_index("c")
        s = lax.axis_index("s")
        w = s * NC + c
        dstb = (db0, db1, db2, db3, db4, db5, db6, db7)
        idsem = (id0, id1, id2, id3, id4, id5, id6, id7)
        scs = (sa, sb)

        def d_sl(j):
            return dst_hbm.at[pl.ds((w * nb + j) * BLK, BLK)]

        def sct(u):
            return pltpu.make_async_copy(ones_v, acc_sh.at[dstb[u % 8]], scs[u % 2])

        pltpu.sync_copy(zeros_hbm, buf_v)

        def zloop(i, _):
            pltpu.sync_copy(buf_v, acc_sh.at[pl.ds(s * ZR + i * ZCH, ZCH)])
            return 0
        lax.fori_loop(0, ZR // ZCH, zloop, 0)
        pltpu.sync_copy(ones_hbm, ones_v)
        plsc.subcore_barrier()

        for j in range(4):
            pltpu.async_copy(d_sl(j), dstb[j], idsem[j])

        def eloop(g, _):
            for u in range(8):
                j = g * 8 + u
                pltpu.make_async_copy(d_sl(j), dstb[u], idsem[u]).wait()
                pltpu.async_copy(ones_v, acc_sh.at[dstb[u]], scs[u % 2], add=True)
                if u > 0:
                    sct(u - 1).wait()
                else:
                    @pl.when(j >= 1)
                    def _():
                        sct(u - 1).wait()

                @pl.when(j + 4 < nb)
                def _():
                    pltpu.async_copy(d_sl(j + 4), dstb[(u + 4) % 8],
                                     idsem[(u + 4) % 8])
            return 0
        lax.fori_loop(0, nb // 8, eloop, 0)
        sct((nb - 1) % 8).wait()
        plsc.subcore_barrier()

        def wloop(i, _):
            r = s * ZR + i * ZCH
            pltpu.sync_copy(acc_sh.at[pl.ds(r, ZCH)], buf_v)
            pltpu.sync_copy(buf_v, out_hbm.at[c, pl.ds(r, ZCH)])
            return 0
        lax.fori_loop(0, ZR // ZCH, wloop, 0)

    return pl.kernel(
        body,
        mesh=_mesh(),
        out_type=jax.ShapeDtypeStruct((NC, N_PAD, D), _f32),
        scratch_types=(
            [pltpu.VMEM((BLK,), jnp.int32)] * 8
            + [pltpu.VMEM((ZCH, D), _f32), pltpu.VMEM((BLK, D), _f32)]
            + [pltpu.SemaphoreType.DMA] * 10
            + [pltpu.VMEM_SHARED((N_PAD, D), _f32)]
        ),
    )


PZ = 80


def _sc_propagate(nb):
    def body(y_hbm, src_hbm, dst_hbm, zeros_hbm, out_hbm,
             sb0, sb1, sb2, sb3, db0, db1, db2, db3, db4, db5, db6, db7,
             r0, r1, r2,
             is0, is1, is2, is3, id0, id1, id2, id3, id4, id5, id6, id7,
             g0, g1, g2, sa, sb, acc_sh):
        c = lax.axis_index("c")
        s = lax.axis_index("s")
        w = s * NC + c
        rows = (r0, r1, r2)
        gsem = (g0, g1, g2)
        srcb = (sb0, sb1, sb2, sb3)
        dstb = (db0, db1, db2, db3, db4, db5, db6, db7)
        isem = (is0, is1, is2, is3)
        idsem = (id0, id1, id2, id3, id4, id5, id6, id7)
        scs = (sa, sb)

        def s_sl(j):
            return src_hbm.at[pl.ds((w * nb + j) * BLK, BLK)]

        def d_sl(j):
            return dst_hbm.at[pl.ds((w * nb + j) * BLK, BLK)]

        def gat(u):
            return pltpu.make_async_copy(
                y_hbm.at[srcb[u % 4]], rows[u % 3], gsem[u % 3])

        def sct(u):
            return pltpu.make_async_copy(
                rows[u % 3], acc_sh.at[dstb[u % 8]], scs[u % 2])

        pltpu.sync_copy(zeros_hbm, r0.at[pl.ds(0, PZ)])

        def zloop(i, _):
            pltpu.sync_copy(r0.at[pl.ds(0, PZ)],
                            acc_sh.at[pl.ds(s * ZR + i * PZ, PZ)])
            return 0
        lax.fori_loop(0, ZR // PZ, zloop, 0)
        plsc.subcore_barrier()

        for j in range(4):
            pltpu.async_copy(s_sl(j), srcb[j], isem[j])
            pltpu.async_copy(d_sl(j), dstb[j], idsem[j])
        for j in range(2):
            pltpu.make_async_copy(s_sl(j), srcb[j], isem[j]).wait()
            pltpu.async_copy(y_hbm.at[srcb[j]], rows[j], gsem[j])

        def eloop(g, _):
            for u in range(24):
                j = g * 24 + u
                gat(u).wait()
                pltpu.make_async_copy(d_sl(j), dstb[u % 8], idsem[u % 8]).wait()
                pltpu.async_copy(rows[u % 3], acc_sh.at[dstb[u % 8]],
                                 scs[u % 2], add=True)
                if u > 0:
                    sct(u - 1).wait()
                else:
                    @pl.when(j >= 1)
                    def _():
                        sct(u - 1).wait()

                @pl.when(j + 2 < nb)
                def _():
                    pltpu.make_async_copy(
                        s_sl(j + 2), srcb[(u + 2) % 4], isem[(u + 2) % 4]).wait()
                    pltpu.async_copy(y_hbm.at[srcb[(u + 2) % 4]],
                                     rows[(u + 2) % 3], gsem[(u + 2) % 3])

                @pl.when(j + 4 < nb)
                def _():
                    pltpu.async_copy(s_sl(j + 4), srcb[u % 4], isem[u % 4])
                    pltpu.async_copy(d_sl(j + 4), dstb[(u + 4) % 8],
                                     idsem[(u + 4) % 8])
            return 0
        lax.fori_loop(0, nb // 24, eloop, 0)
        sct((nb - 1) % 24).wait()
        plsc.subcore_barrier()

        def wloop(i, _):
            r = s * ZR + i * PZ
            pltpu.sync_copy(acc_sh.at[pl.ds(r, PZ)], r0.at[pl.ds(0, PZ)])
            pltpu.sync_copy(r0.at[pl.ds(0, PZ)], out_hbm.at[c, pl.ds(r, PZ)])
            return 0
        lax.fori_loop(0, ZR // PZ, wloop, 0)

    return pl.kernel(
        body,
        mesh=_mesh(),
        out_type=jax.ShapeDtypeStruct((NC, N_PAD, D), _f32),
        scratch_types=(
            [pltpu.VMEM((BLK,), jnp.int32)] * 12
            + [pltpu.VMEM((BLK, D), _f32)] * 3
            + [pltpu.SemaphoreType.DMA] * 17
            + [pltpu.VMEM_SHARED((N_PAD, D), _f32)]
        ),
    )


RB = 2048


def _dinv_of(d0, d1):
    deg = d0[:, 0:1] + d1[:, 0:1]
    return jnp.where(deg > 0.0, lax.rsqrt(deg), 0.0)


def _tc_y1(x_ref, w_ref, d0_ref, d1_ref, y_ref):
    dinv = _dinv_of(d0_ref[...], d1_ref[...])
    xw = jnp.dot(x_ref[...], w_ref[...], preferred_element_type=_f32)
    y_ref[...] = xw * dinv


def _tc_y2(a0_ref, a1_ref, d0_ref, d1_ref, b1_ref, g_ref, be_ref, w2_ref, y_ref):
    dinv = _dinv_of(d0_ref[...], d1_ref[...])
    h = (a0_ref[...] + a1_ref[...]) * dinv + b1_ref[...]
    h = h * (g_ref[...] * INV_SQRT1P) + be_ref[...]
    h = jnp.maximum(h, 0.0)
    y_ref[...] = jnp.dot(h, w2_ref[...], preferred_element_type=_f32) * dinv


def _tc_out(a0_ref, a1_ref, d0_ref, d1_ref, b2_ref, o_ref):
    dinv = _dinv_of(d0_ref[...], d1_ref[...])
    o_ref[...] = (a0_ref[...] + a1_ref[...]) * dinv + b2_ref[...]


def _row_spec(i_dim):
    return pl.BlockSpec((RB, i_dim), lambda i: (i, 0))


def _bcast_spec(shape):
    return pl.BlockSpec(shape, lambda i: (0, 0))


def kernel(x, edge_index, W1, b1, gamma1, beta1, W2, b2):
    e = edge_index.shape[1]
    e_tot = e + N
    nb = _nb(e_tot)
    nb = ((nb + 23) // 24) * 24
    e_pad = NW * nb * BLK - e_tot

    loop = jnp.arange(N, dtype=jnp.int32)
    padi = jnp.arange(e_pad, dtype=jnp.int32)
    src_f = jnp.concatenate([edge_index[0].astype(jnp.int32), loop, N + (padi % 64)])
    dst_f = jnp.concatenate([edge_index[1].astype(jnp.int32), loop, N + 64 + (padi % 64)])
    x_pad = jnp.pad(x, ((0, N_PAD - N), (0, 0)))

    zerosD = jnp.zeros((ZCH, D), _f32)
    onesD = jnp.ones((BLK, D), _f32)
    zerosP = jnp.zeros((PZ, D), _f32)

    deg = _sc_degree(nb)(dst_f, zerosD, onesD)
    d0, d1 = deg[0], deg[1]

    grid = (N_PAD // RB,)
    y1 = pl.pallas_call(
        _tc_y1,
        grid=grid,
        in_specs=[_row_spec(D), _bcast_spec((D, D)), _row_spec(D), _row_spec(D)],
        out_specs=_row_spec(D),
        out_shape=jax.ShapeDtypeStruct((N_PAD, D), _f32),
    )(x_pad, W1, d0, d1)

    prop = _sc_propagate(nb)
    acc1 = prop(y1, src_f, dst_f, zerosP)

    y2 = pl.pallas_call(
        _tc_y2,
        grid=grid,
        in_specs=[_row_spec(D), _row_spec(D), _row_spec(D), _row_spec(D),
                  _bcast_spec((1, D)), _bcast_spec((1, D)), _bcast_spec((1, D)),
                  _bcast_spec((D, D))],
        out_specs=_row_spec(D),
        out_shape=jax.ShapeDtypeStruct((N_PAD, D), _f32),
    )(acc1[0], acc1[1], d0, d1, b1.reshape(1, D), gamma1.reshape(1, D),
      beta1.reshape(1, D), W2)

    acc2 = prop(y2, src_f, dst_f, zerosP)

    out = pl.pallas_call(
        _tc_out,
        grid=grid,
        in_specs=[_row_spec(D), _row_spec(D), _row_spec(D), _row_spec(D),
                  _bcast_spec((1, D))],
        out_specs=_row_spec(D),
        out_shape=jax.ShapeDtypeStruct((N_PAD, D), _f32),
    )(acc2[0], acc2[1], d0, d1, b2.reshape(1, D))

    return out[:N]


# speedup vs baseline: 26.5479x; 1.0128x over previous
import functools

import jax
import jax.numpy as jnp
from jax import lax
from jax.experimental import pallas as pl
from jax.experimental.pallas import tpu as pltpu
from jax.experimental.pallas import tpu_sc as plsc

N = 10000
D = 128
NC = 2
NS = 16
NW = NC * NS
BLK = 112
N_PAD = 10240
ZR = N_PAD // NS
ZCH = 128
EPS = 1e-5
INV_SQRT1P = 1.0 / (1.0 + EPS) ** 0.5

_f32 = jnp.float32


@functools.cache
def _mesh():
    return plsc.VectorSubcoreMesh(
        core_axis_name="c", subcore_axis_name="s", num_cores=NC, num_subcores=NS)


def _nb(e_tot):
    return (e_tot + NW * BLK - 1) // (NW * BLK)


def _sc_degree(nb):
    def body(dst_hbm, zeros_hbm, ones_hbm, out_hbm,
             db0, db1, db2, db3, db4, db5, db6, db7, buf_v, ones_v,
             id0, id1, id2, id3, id4, id5, id6, id7, sa, sb, acc_sh):
        c = lax.axis_index("c")
        s = lax.axis_index("s")
        w = s * NC + c
        dstb = (db0, db1, db2, db3, db4, db5, db6, db7)
        idsem = (id0, id1, id2, id3, id4, id5, id6, id7)
        scs = (sa, sb)

        def d_sl(j):
            return dst_hbm.at[pl.ds((w * nb + j) * BLK, BLK)]

        def sct(u):
            return pltpu.make_async_copy(ones_v, acc_sh.at[dstb[u % 8]], scs[u % 2])

        pltpu.sync_copy(zeros_hbm, buf_v)

        def zloop(i, _):
            pltpu.sync_copy(buf_v, acc_sh.at[pl.ds(s * ZR + i * ZCH, ZCH)])
            return 0
        lax.fori_loop(0, ZR // ZCH, zloop, 0)
        pltpu.sync_copy(ones_hbm, ones_v)
        plsc.subcore_barrier()

        for j in range(4):
            pltpu.async_copy(d_sl(j), dstb[j], idsem[j])

        def eloop(g, _):
            for u in range(8):
                j = g * 8 + u
                pltpu.make_async_copy(d_sl(j), dstb[u], idsem[u]).wait()
                pltpu.async_copy(ones_v, acc_sh.at[dstb[u]], scs[u % 2], add=True)
                if u > 0:
                    sct(u - 1).wait()
                else:
                    @pl.when(j >= 1)
                    def _():
                        sct(u - 1).wait()

                @pl.when(j + 4 < nb)
                def _():
                    pltpu.async_copy(d_sl(j + 4), dstb[(u + 4) % 8],
                                     idsem[(u + 4) % 8])
            return 0
        lax.fori_loop(0, nb // 8, eloop, 0)
        sct((nb - 1) % 8).wait()
        plsc.subcore_barrier()

        def wloop(i, _):
            r = s * ZR + i * ZCH
            pltpu.sync_copy(acc_sh.at[pl.ds(r, ZCH)], buf_v)
            pltpu.sync_copy(buf_v, out_hbm.at[c, pl.ds(r, ZCH)])
            return 0
        lax.fori_loop(0, ZR // ZCH, wloop, 0)

    return pl.kernel(
        body,
        mesh=_mesh(),
        out_type=jax.ShapeDtypeStruct((NC, N_PAD, D), _f32),
        scratch_types=(
            [pltpu.VMEM((BLK,), jnp.int32)] * 8
            + [pltpu.VMEM((ZCH, D), _f32), pltpu.VMEM((BLK, D), _f32)]
            + [pltpu.SemaphoreType.DMA] * 10
            + [pltpu.VMEM_SHARED((N_PAD, D), _f32)]
        ),
    )


PZ = 80


def _sc_propagate(nb):
    def body(y_hbm, src_hbm, dst_hbm, zeros_hbm, out_hbm,
             sb0, sb1, sb2, sb3, db0, db1, db2, db3, db4, db5, db6, db7,
             r0, r1, r2,
             is0, is1, is2, is3, id0, id1, id2, id3, id4, id5, id6, id7,
             g0, g1, g2, sa, sb, acc_sh):
        c = lax.axis_index("c")
        s = lax.axis_index("s")
        w = s * NC + c
        rows = (r0, r1, r2)
        gsem = (g0, g1, g2)
        srcb = (sb0, sb1, sb2, sb3)
        dstb = (db0, db1, db2, db3, db4, db5, db6, db7)
        isem = (is0, is1, is2, is3)
        idsem = (id0, id1, id2, id3, id4, id5, id6, id7)
        scs = (sa, sb)

        def s_sl(j):
            return src_hbm.at[pl.ds((w * nb + j) * BLK, BLK)]

        def d_sl(j):
            return dst_hbm.at[pl.ds((w * nb + j) * BLK, BLK)]

        def gat(u):
            return pltpu.make_async_copy(
                y_hbm.at[srcb[u % 4]], rows[u % 3], gsem[u % 3])

        def sct(u):
            return pltpu.make_async_copy(
                rows[u % 3], acc_sh.at[dstb[u % 8]], scs[u % 2])

        pltpu.sync_copy(zeros_hbm, r0.at[pl.ds(0, PZ)])

        def zloop(i, _):
            pltpu.sync_copy(r0.at[pl.ds(0, PZ)],
                            acc_sh.at[pl.ds(s * ZR + i * PZ, PZ)])
            return 0
        lax.fori_loop(0, ZR // PZ, zloop, 0)
        plsc.subcore_barrier()

        for j in range(4):
            pltpu.async_copy(s_sl(j), srcb[j], isem[j])
            pltpu.async_copy(d_sl(j), dstb[j], idsem[j])
        for j in range(2):
            pltpu.make_async_copy(s_sl(j), srcb[j], isem[j]).wait()
            pltpu.async_copy(y_hbm.at[srcb[j]], rows[j], gsem[j])

        def eloop(g, _):
            for u in range(24):
                j = g * 24 + u
                gat(u).wait()
                pltpu.make_async_copy(d_sl(j), dstb[u % 8], idsem[u % 8]).wait()
                pltpu.async_copy(rows[u % 3], acc_sh.at[dstb[u % 8]],
                                 scs[u % 2], add=True)
                if u > 0:
                    sct(u - 1).wait()
                else:
                    @pl.when(j >= 1)
                    def _():
                        sct(u - 1).wait()

                @pl.when(j + 2 < nb)
                def _():
                    pltpu.make_async_copy(
                        s_sl(j + 2), srcb[(u + 2) % 4], isem[(u + 2) % 4]).wait()
                    pltpu.async_copy(y_hbm.at[srcb[(u + 2) % 4]],
                                     rows[(u + 2) % 3], gsem[(u + 2) % 3])

                @pl.when(j + 4 < nb)
                def _():
                    pltpu.async_copy(s_sl(j + 4), srcb[u % 4], isem[u % 4])
                    pltpu.async_copy(d_sl(j + 4), dstb[(u + 4) % 8],
                                     idsem[(u + 4) % 8])
            return 0
        lax.fori_loop(0, nb // 24, eloop, 0)
        sct((nb - 1) % 24).wait()
        plsc.subcore_barrier()

        def wloop(i, _):
            r = s * ZR + i * PZ
            pltpu.sync_copy(acc_sh.at[pl.ds(r, PZ)], r0.at[pl.ds(0, PZ)])
            pltpu.sync_copy(r0.at[pl.ds(0, PZ)], out_hbm.at[c, pl.ds(r, PZ)])
            return 0
        lax.fori_loop(0, ZR // PZ, wloop, 0)

    return pl.kernel(
        body,
        mesh=_mesh(),
        out_type=jax.ShapeDtypeStruct((NC, N_PAD, D), _f32),
        scratch_types=(
            [pltpu.VMEM((BLK,), jnp.int32)] * 12
            + [pltpu.VMEM((BLK, D), _f32)] * 3
            + [pltpu.SemaphoreType.DMA] * 17
            + [pltpu.VMEM_SHARED((N_PAD, D), _f32)]
        ),
    )


RB = 2048
RBO = 2000


def _tc_dinv(d0_ref, d1_ref, o_ref):
    deg = d0_ref[:, 0:1] + d1_ref[:, 0:1]
    dinv = jnp.where(deg > 0.0, lax.rsqrt(deg), 0.0)
    o_ref[...] = jnp.broadcast_to(dinv, o_ref.shape)


def _tc_y1(x_ref, w_ref, dv_ref, y_ref):
    xw = jnp.dot(x_ref[...], w_ref[...], preferred_element_type=_f32)
    y_ref[...] = xw * dv_ref[:, 0:1]


def _tc_y2(a0_ref, a1_ref, yp_ref, dv_ref, b1_ref, g_ref, be_ref, w2_ref, y_ref):
    dinv = dv_ref[:, 0:1]
    h = (a0_ref[...] + a1_ref[...] + yp_ref[...]) * dinv + b1_ref[...]
    h = h * (g_ref[...] * INV_SQRT1P) + be_ref[...]
    h = jnp.maximum(h, 0.0)
    y_ref[...] = jnp.dot(h, w2_ref[...], preferred_element_type=_f32) * dinv


def _tc_out(a0_ref, a1_ref, yp_ref, dv_ref, b2_ref, o_ref):
    o_ref[...] = ((a0_ref[...] + a1_ref[...] + yp_ref[...]) * dv_ref[:, 0:1]
                  + b2_ref[...])


def _rspec(rows, i_dim):
    return pl.BlockSpec((rows, i_dim), lambda i: (i, 0))


def _row_spec(i_dim):
    return pl.BlockSpec((RB, i_dim), lambda i: (i, 0))


def _bcast_spec(shape):
    return pl.BlockSpec(shape, lambda i: (0, 0))


def kernel(x, edge_index, W1, b1, gamma1, beta1, W2, b2):
    e = edge_index.shape[1]
    nb = _nb(e)
    nb = ((nb + 23) // 24) * 24
    e_pad = NW * nb * BLK - e

    padi = jnp.arange(e_pad, dtype=jnp.int32)
    src_f = jnp.concatenate([edge_index[0].astype(jnp.int32), N + (padi % 64)])
    dst_f = jnp.concatenate([edge_index[1].astype(jnp.int32), N + 64 + (padi % 64)])
    x_pad = jnp.pad(x, ((0, N_PAD - N), (0, 0)))

    halves = jnp.full((ZCH, D), 0.5, _f32)
    onesD = jnp.ones((BLK, D), _f32)
    zerosP = jnp.zeros((PZ, D), _f32)

    deg = _sc_degree(nb)(dst_f, halves, onesD)

    grid = (N_PAD // RB,)
    dinv8 = pl.pallas_call(
        _tc_dinv,
        grid=grid,
        in_specs=[_row_spec(D), _row_spec(D)],
        out_specs=_row_spec(8),
        out_shape=jax.ShapeDtypeStruct((N_PAD, 8), _f32),
    )(deg[0], deg[1])

    y1 = pl.pallas_call(
        _tc_y1,
        grid=grid,
        in_specs=[_row_spec(D), _bcast_spec((D, D)), _row_spec(8)],
        out_specs=_row_spec(D),
        out_shape=jax.ShapeDtypeStruct((N_PAD, D), _f32),
    )(x_pad, W1, dinv8)

    prop = _sc_propagate(nb)
    acc1 = prop(y1, src_f, dst_f, zerosP)

    y2 = pl.pallas_call(
        _tc_y2,
        grid=grid,
        in_specs=[_row_spec(D), _row_spec(D), _row_spec(D), _row_spec(8),
                  _bcast_spec((1, D)), _bcast_spec((1, D)), _bcast_spec((1, D)),
                  _bcast_spec((D, D))],
        out_specs=_row_spec(D),
        out_shape=jax.ShapeDtypeStruct((N_PAD, D), _f32),
    )(acc1[0], acc1[1], y1, dinv8, b1.reshape(1, D), gamma1.reshape(1, D),
      beta1.reshape(1, D), W2)

    acc2 = prop(y2, src_f, dst_f, zerosP)

    out = pl.pallas_call(
        _tc_out,
        grid=(N // RBO,),
        in_specs=[_rspec(RBO, D), _rspec(RBO, D), _rspec(RBO, D), _rspec(RBO, 8),
                  _bcast_spec((1, D))],
        out_specs=_rspec(RBO, D),
        out_shape=jax.ShapeDtypeStruct((N, D), _f32),
    )(acc2[0], acc2[1], y2, dinv8, b2.reshape(1, D))

    return out


# speedup vs baseline: 26.9703x; 1.0159x over previous
import functools

import jax
import jax.numpy as jnp
from jax import lax
from jax.experimental import pallas as pl
from jax.experimental.pallas import tpu as pltpu
from jax.experimental.pallas import tpu_sc as plsc

N = 10000
D = 128
NC = 2
NS = 16
NW = NC * NS
BLK = 112
N_PAD = 10240
ZR = N_PAD // NS
ZCH = 128
EPS = 1e-5
INV_SQRT1P = 1.0 / (1.0 + EPS) ** 0.5

_f32 = jnp.float32


@functools.cache
def _mesh():
    return plsc.VectorSubcoreMesh(
        core_axis_name="c", subcore_axis_name="s", num_cores=NC, num_subcores=NS)


def _nb(e_tot):
    return (e_tot + NW * BLK - 1) // (NW * BLK)


def _sc_degree(nb):
    def body(dst_hbm, zeros_hbm, ones_hbm, out_hbm,
             db0, db1, db2, db3, db4, db5, db6, db7, buf_v, ones_v,
             id0, id1, id2, id3, id4, id5, id6, id7, sa, sb, acc_sh):
        c = lax.axis_index("c")
        s = lax.axis_index("s")
        w = s * NC + c
        dstb = (db0, db1, db2, db3, db4, db5, db6, db7)
        idsem = (id0, id1, id2, id3, id4, id5, id6, id7)
        scs = (sa, sb)

        def d_sl(j):
            return dst_hbm.at[pl.ds((w * nb + j) * BLK, BLK)]

        def sct(u):
            return pltpu.make_async_copy(ones_v, acc_sh.at[dstb[u % 8]], scs[u % 2])

        pltpu.sync_copy(zeros_hbm, buf_v)

        def zloop(i, _):
            pltpu.sync_copy(buf_v, acc_sh.at[pl.ds(s * ZR + i * ZCH, ZCH)])
            return 0
        lax.fori_loop(0, ZR // ZCH, zloop, 0)
        pltpu.sync_copy(ones_hbm, ones_v)
        plsc.subcore_barrier()

        for j in range(4):
            pltpu.async_copy(d_sl(j), dstb[j], idsem[j])

        def eloop(g, _):
            for u in range(8):
                j = g * 8 + u
                pltpu.make_async_copy(d_sl(j), dstb[u], idsem[u]).wait()
                pltpu.async_copy(ones_v, acc_sh.at[dstb[u]], scs[u % 2], add=True)
                if u > 0:
                    sct(u - 1).wait()
                else:
                    @pl.when(j >= 1)
                    def _():
                        sct(u - 1).wait()

                @pl.when(j + 4 < nb)
                def _():
                    pltpu.async_copy(d_sl(j + 4), dstb[(u + 4) % 8],
                                     idsem[(u + 4) % 8])
            return 0
        lax.fori_loop(0, nb // 8, eloop, 0)
        sct((nb - 1) % 8).wait()
        plsc.subcore_barrier()

        def wloop(i, _):
            r = s * ZR + i * ZCH
            pltpu.sync_copy(acc_sh.at[pl.ds(r, ZCH)], buf_v)
            pltpu.sync_copy(buf_v, out_hbm.at[c, pl.ds(r, ZCH)])
            return 0
        lax.fori_loop(0, ZR // ZCH, wloop, 0)

    return pl.kernel(
        body,
        mesh=_mesh(),
        out_type=jax.ShapeDtypeStruct((NC, N_PAD, D), _f32),
        scratch_types=(
            [pltpu.VMEM((BLK,), jnp.int32)] * 8
            + [pltpu.VMEM((ZCH, D), _f32), pltpu.VMEM((BLK, D), _f32)]
            + [pltpu.SemaphoreType.DMA] * 10
            + [pltpu.VMEM_SHARED((N_PAD, D), _f32)]
        ),
    )


PZ = 80


def _sc_propagate(nb):
    def body(y_hbm, src_hbm, dst_hbm, zeros_hbm, out_hbm,
             sb0, sb1, sb2, sb3, db0, db1, db2, db3, db4, db5, db6, db7,
             r0, r1, r2,
             is0, is1, is2, is3, id0, id1, id2, id3, id4, id5, id6, id7,
             g0, g1, g2, sa, sb, acc_sh):
        c = lax.axis_index("c")
        s = lax.axis_index("s")
        w = s * NC + c
        rows = (r0, r1, r2)
        gsem = (g0, g1, g2)
        srcb = (sb0, sb1, sb2, sb3)
        dstb = (db0, db1, db2, db3, db4, db5, db6, db7)
        isem = (is0, is1, is2, is3)
        idsem = (id0, id1, id2, id3, id4, id5, id6, id7)
        scs = (sa, sb)

        def s_sl(j):
            return src_hbm.at[pl.ds((w * nb + j) * BLK, BLK)]

        def d_sl(j):
            return dst_hbm.at[pl.ds((w * nb + j) * BLK, BLK)]

        def gat(u):
            return pltpu.make_async_copy(
                y_hbm.at[srcb[u % 4]], rows[u % 3], gsem[u % 3])

        def sct(u):
            return pltpu.make_async_copy(
                rows[u % 3], acc_sh.at[dstb[u % 8]], scs[u % 2])

        pltpu.sync_copy(zeros_hbm, r0.at[pl.ds(0, PZ)])

        def zloop(i, _):
            pltpu.sync_copy(r0.at[pl.ds(0, PZ)],
                            acc_sh.at[pl.ds(s * ZR + i * PZ, PZ)])
            return 0
        lax.fori_loop(0, ZR // PZ, zloop, 0)
        plsc.subcore_barrier()

        for j in range(4):
            pltpu.async_copy(s_sl(j), srcb[j], isem[j])
            pltpu.async_copy(d_sl(j), dstb[j], idsem[j])
        for j in range(2):
            pltpu.make_async_copy(s_sl(j), srcb[j], isem[j]).wait()
            pltpu.async_copy(y_hbm.at[srcb[j]], rows[j], gsem[j])

        def eloop(g, _):
            for u in range(24):
                j = g * 24 + u
                gat(u).wait()
                pltpu.make_async_copy(d_sl(j), dstb[u % 8], idsem[u % 8]).wait()
                pltpu.async_copy(rows[u % 3], acc_sh.at[dstb[u % 8]],
                                 scs[u % 2], add=True)
                if u > 0:
                    sct(u - 1).wait()
                else:
                    @pl.when(j >= 1)
                    def _():
                        sct(u - 1).wait()

                @pl.when(j + 2 < nb)
                def _():
                    pltpu.make_async_copy(
                        s_sl(j + 2), srcb[(u + 2) % 4], isem[(u + 2) % 4]).wait()
                    pltpu.async_copy(y_hbm.at[srcb[(u + 2) % 4]],
                                     rows[(u + 2) % 3], gsem[(u + 2) % 3])

                @pl.when(j + 4 < nb)
                def _():
                    pltpu.async_copy(s_sl(j + 4), srcb[u % 4], isem[u % 4])
                    pltpu.async_copy(d_sl(j + 4), dstb[(u + 4) % 8],
                                     idsem[(u + 4) % 8])
            return 0
        lax.fori_loop(0, nb // 24, eloop, 0)
        sct((nb - 1) % 24).wait()
        plsc.subcore_barrier()

        def wloop(i, _):
            r = s * ZR + i * PZ
            pltpu.sync_copy(acc_sh.at[pl.ds(r, PZ)], r0.at[pl.ds(0, PZ)])
            pltpu.sync_copy(r0.at[pl.ds(0, PZ)], out_hbm.at[c, pl.ds(r, PZ)])
            return 0
        lax.fori_loop(0, ZR // PZ, wloop, 0)

    return pl.kernel(
        body,
        mesh=_mesh(),
        out_type=jax.ShapeDtypeStruct((NC, N_PAD, D), _f32),
        scratch_types=(
            [pltpu.VMEM((BLK,), jnp.int32)] * 12
            + [pltpu.VMEM((BLK, D), _f32)] * 3
            + [pltpu.SemaphoreType.DMA] * 17
            + [pltpu.VMEM_SHARED((N_PAD, D), _f32)]
        ),
    )


RB = 2048
RBO = 2000


def _tc_y1(x_ref, w_ref, d0_ref, d1_ref, y_ref, dv_ref):
    deg = d0_ref[:, 0:1] + d1_ref[:, 0:1]
    dinv = jnp.where(deg > 0.0, lax.rsqrt(deg), 0.0)
    dv_ref[...] = jnp.broadcast_to(dinv, dv_ref.shape)
    xw = jnp.dot(x_ref[...], w_ref[...], preferred_element_type=_f32)
    y_ref[...] = xw * dinv


def _tc_y2(a0_ref, a1_ref, yp_ref, dv_ref, b1_ref, g_ref, be_ref, w2_ref, y_ref):
    dinv = dv_ref[:, 0:1]
    h = (a0_ref[...] + a1_ref[...] + yp_ref[...]) * dinv + b1_ref[...]
    h = h * (g_ref[...] * INV_SQRT1P) + be_ref[...]
    h = jnp.maximum(h, 0.0)
    y_ref[...] = jnp.dot(h, w2_ref[...], preferred_element_type=_f32) * dinv


def _tc_out(a0_ref, a1_ref, yp_ref, dv_ref, b2_ref, o_ref):
    o_ref[...] = ((a0_ref[...] + a1_ref[...] + yp_ref[...]) * dv_ref[:, 0:1]
                  + b2_ref[...])


def _rspec(rows, i_dim):
    return pl.BlockSpec((rows, i_dim), lambda i: (i, 0))


def _row_spec(i_dim):
    return pl.BlockSpec((RB, i_dim), lambda i: (i, 0))


def _bcast_spec(shape):
    return pl.BlockSpec(shape, lambda i: (0, 0))


def kernel(x, edge_index, W1, b1, gamma1, beta1, W2, b2):
    e = edge_index.shape[1]
    nb = _nb(e)
    nb = ((nb + 23) // 24) * 24
    e_pad = NW * nb * BLK - e

    padi = jnp.arange(e_pad, dtype=jnp.int32)
    src_f = jnp.concatenate([edge_index[0].astype(jnp.int32), N + (padi % 64)])
    dst_f = jnp.concatenate([edge_index[1].astype(jnp.int32), N + 64 + (padi % 64)])
    x_pad = jnp.pad(x, ((0, N_PAD - N), (0, 0)))

    halves = jnp.full((ZCH, D), 0.5, _f32)
    onesD = jnp.ones((BLK, D), _f32)
    zerosP = jnp.zeros((PZ, D), _f32)

    deg = _sc_degree(nb)(dst_f, halves, onesD)

    grid = (N_PAD // RB,)
    y1, dinv8 = pl.pallas_call(
        _tc_y1,
        grid=grid,
        in_specs=[_row_spec(D), _bcast_spec((D, D)), _row_spec(D), _row_spec(D)],
        out_specs=(_row_spec(D), _row_spec(8)),
        out_shape=(jax.ShapeDtypeStruct((N_PAD, D), _f32),
                   jax.ShapeDtypeStruct((N_PAD, 8), _f32)),
    )(x_pad, W1, deg[0], deg[1])

    prop = _sc_propagate(nb)
    acc1 = prop(y1, src_f, dst_f, zerosP)

    y2 = pl.pallas_call(
        _tc_y2,
        grid=grid,
        in_specs=[_row_spec(D), _row_spec(D), _row_spec(D), _row_spec(8),
                  _bcast_spec((1, D)), _bcast_spec((1, D)), _bcast_spec((1, D)),
                  _bcast_spec((D, D))],
        out_specs=_row_spec(D),
        out_shape=jax.ShapeDtypeStruct((N_PAD, D), _f32),
    )(acc1[0], acc1[1], y1, dinv8, b1.reshape(1, D), gamma1.reshape(1, D),
      beta1.reshape(1, D), W2)

    acc2 = prop(y2, src_f, dst_f, zerosP)

    out = pl.pallas_call(
        _tc_out,
        grid=(N // RBO,),
        in_specs=[_rspec(RBO, D), _rspec(RBO, D), _rspec(RBO, D), _rspec(RBO, 8),
                  _bcast_spec((1, D))],
        out_specs=_rspec(RBO, D),
        out_shape=jax.ShapeDtypeStruct((N, D), _f32),
    )(acc2[0], acc2[1], y2, dinv8, b2.reshape(1, D))

    return out
